# baseline (speedup 1.0000x reference)
import functools

import jax
import jax.numpy as jnp
from jax.experimental import pallas as pl
from jax.experimental.pallas import tpu as pltpu

_NCH = 4
_HS = 128


def _cell_kernel(x_ref, xm_ref, nh_ref, nc_ref, pos_ref,
                 wfin_ref, bfin_ref, wf_ref, bf_ref, wa_ref, ba_ref,
                 h_ref, c_ref):
    x = x_ref[...]
    xm = xm_ref[...]
    nh = nh_ref[...]
    nc = nc_ref[...]
    pos = pos_ref[...]

    f_in = (jnp.dot(x.astype(jnp.bfloat16), wfin_ref[...],
                    preferred_element_type=jnp.float32)
            + bfin_ref[...]) * xm

    h_ch = [nh[:, k * _HS:(k + 1) * _HS] for k in range(_NCH)]
    c_ch = [nc[:, k * _HS:(k + 1) * _HS] for k in range(_NCH)]
    nh_cols = []
    nc_cols = []
    for j in range(_NCH):
        pj = pos[:, j][:, None]
        hj = jnp.where(pj == 0, h_ch[0],
             jnp.where(pj == 1, h_ch[1],
             jnp.where(pj == 2, h_ch[2], h_ch[3])))
        cj = jnp.where(pj == 0, c_ch[0],
             jnp.where(pj == 1, c_ch[1],
             jnp.where(pj == 2, c_ch[2], c_ch[3])))
        nh_cols.append(hj)
        nc_cols.append(cj)
    nh_perm = jnp.concatenate(nh_cols, axis=1).astype(jnp.bfloat16)

    fg = jnp.dot(nh_perm, wf_ref[...],
                 preferred_element_type=jnp.float32) + bf_ref[...]
    iou = jnp.dot(nh_perm, wa_ref[...],
                  preferred_element_type=jnp.float32) + ba_ref[...]

    two_f_in = 2.0 * f_in
    c = jnp.zeros_like(f_in)
    for j in range(_NCH):
        f_j = jax.nn.sigmoid(fg[:, j * _HS:(j + 1) * _HS] + two_f_in)
        c = c + f_j * nc_cols[j]

    h_ref[...] = iou * jnp.tanh(c)
    c_ref[...] = c


@functools.partial(jax.jit, static_argnames=())
def kernel(x, x_mask, neighbour_h, neighbour_c, pos,
           W_fin, b_fin, W_f, b_f, W_aggr, b_aggr):
    n, xs = x.shape
    _, nch, hs = neighbour_h.shape
    fw = nch * hs

    block = 2000
    grid = (pl.cdiv(n, block),)

    nh_flat = neighbour_h.reshape(n, fw)
    nc_flat = neighbour_c.reshape(n, fw)
    xm2 = x_mask.reshape(n, 1)

    row = lambda i: (i, 0)
    rep = lambda i: (0, 0)

    h, c = pl.pallas_call(
        _cell_kernel,
        grid=grid,
        in_specs=[
            pl.BlockSpec((block, xs), row),
            pl.BlockSpec((block, 1), row),
            pl.BlockSpec((block, fw), row),
            pl.BlockSpec((block, fw), row),
            pl.BlockSpec((block, nch), row),
            pl.BlockSpec((xs, hs), rep),
            pl.BlockSpec((1, hs), rep),
            pl.BlockSpec((fw, fw), rep),
            pl.BlockSpec((1, fw), rep),
            pl.BlockSpec((fw, hs), rep),
            pl.BlockSpec((1, hs), rep),
        ],
        out_specs=[
            pl.BlockSpec((block, hs), row),
            pl.BlockSpec((block, hs), row),
        ],
        out_shape=[
            jax.ShapeDtypeStruct((n, hs), jnp.float32),
            jax.ShapeDtypeStruct((n, hs), jnp.float32),
        ],
    )(x, xm2, nh_flat, nc_flat, pos,
      W_fin.astype(jnp.bfloat16), b_fin.reshape(1, hs),
      W_f.astype(jnp.bfloat16), b_f.reshape(1, fw),
      W_aggr.astype(jnp.bfloat16), b_aggr.reshape(1, hs))
    return h, c


# speedup vs baseline: 1.0255x; 1.0255x over previous
import functools

import jax
import jax.numpy as jnp
from jax.experimental import pallas as pl
from jax.experimental.pallas import tpu as pltpu

_NCH = 4
_HS = 128


def _cell_kernel(x_ref, xm_ref, nh_ref, nc_ref, pos_ref,
                 wfin_ref, bfin_ref, wf_ref, bf_ref, wa_ref, ba_ref,
                 h_ref, c_ref):
    x = x_ref[...]
    xm = xm_ref[...]
    nh = nh_ref[...]
    nc = nc_ref[...]
    pos = pos_ref[...]

    f_in = (jnp.dot(x, wfin_ref[...], preferred_element_type=jnp.float32)
            + bfin_ref[...]) * xm

    h_ch = [nh[:, k * _HS:(k + 1) * _HS] for k in range(_NCH)]
    c_ch = [nc[:, k * _HS:(k + 1) * _HS] for k in range(_NCH)]
    nh_cols = []
    nc_cols = []
    for j in range(_NCH):
        pj = pos[:, j][:, None]
        m0 = pj == 0
        m1 = pj == 1
        m2 = pj == 2
        hj = jnp.where(m0, h_ch[0],
             jnp.where(m1, h_ch[1],
             jnp.where(m2, h_ch[2], h_ch[3])))
        cj = jnp.where(m0, c_ch[0],
             jnp.where(m1, c_ch[1],
             jnp.where(m2, c_ch[2], c_ch[3])))
        nh_cols.append(hj)
        nc_cols.append(cj)
    nh_perm = jnp.concatenate(nh_cols, axis=1)

    fg = jnp.dot(nh_perm, wf_ref[...],
                 preferred_element_type=jnp.float32) + bf_ref[...]
    iou = jnp.dot(nh_perm, wa_ref[...],
                  preferred_element_type=jnp.float32) + ba_ref[...]

    two_f_in = 2.0 * f_in
    c = jnp.zeros_like(f_in)
    for j in range(_NCH):
        f_j = jax.nn.sigmoid(fg[:, j * _HS:(j + 1) * _HS] + two_f_in)
        c = c + f_j * nc_cols[j]

    h_ref[...] = iou * jnp.tanh(c)
    c_ref[...] = c


@functools.partial(jax.jit, static_argnames=())
def kernel(x, x_mask, neighbour_h, neighbour_c, pos,
           W_fin, b_fin, W_f, b_f, W_aggr, b_aggr):
    n, xs = x.shape
    _, nch, hs = neighbour_h.shape
    fw = nch * hs

    block = 2000
    grid = (pl.cdiv(n, block),)

    nh_flat = neighbour_h.reshape(n, fw)
    nc_flat = neighbour_c.reshape(n, fw)
    xm2 = x_mask.reshape(n, 1)

    row = lambda i: (i, 0)
    rep = lambda i: (0, 0)

    h, c = pl.pallas_call(
        _cell_kernel,
        grid=grid,
        in_specs=[
            pl.BlockSpec((block, xs), row),
            pl.BlockSpec((block, 1), row),
            pl.BlockSpec((block, fw), row),
            pl.BlockSpec((block, fw), row),
            pl.BlockSpec((block, nch), row),
            pl.BlockSpec((xs, hs), rep),
            pl.BlockSpec((1, hs), rep),
            pl.BlockSpec((fw, fw), rep),
            pl.BlockSpec((1, fw), rep),
            pl.BlockSpec((fw, hs), rep),
            pl.BlockSpec((1, hs), rep),
        ],
        out_specs=[
            pl.BlockSpec((block, hs), row),
            pl.BlockSpec((block, hs), row),
        ],
        out_shape=[
            jax.ShapeDtypeStruct((n, hs), jnp.float32),
            jax.ShapeDtypeStruct((n, hs), jnp.float32),
        ],
    )(x, xm2, nh_flat, nc_flat, pos,
      W_fin, b_fin.reshape(1, hs), W_f, b_f.reshape(1, fw),
      W_aggr, b_aggr.reshape(1, hs))
    return h, c


# speedup vs baseline: 1.1587x; 1.1299x over previous
import functools

import jax
import jax.numpy as jnp
from jax.experimental import pallas as pl
from jax.experimental.pallas import tpu as pltpu

_NCH = 4
_HS = 128


def _cell_kernel(x_ref, xm_ref, nh_ref, nc_ref, pos_ref,
                 wfin_ref, bfin_ref, wf_ref, bf_ref, wa_ref, ba_ref,
                 h_ref, c_ref):
    x = x_ref[...]
    xm = xm_ref[...]
    pos = pos_ref[...]

    f_in = (jnp.dot(x, wfin_ref[...], preferred_element_type=jnp.float32)
            + bfin_ref[...]) * xm

    h_ch = [nh_ref[:, k, :] for k in range(_NCH)]
    c_ch = [nc_ref[:, k, :] for k in range(_NCH)]
    nh_cols = []
    nc_cols = []
    for j in range(_NCH):
        pj = pos[:, j][:, None]
        m0 = pj == 0
        m1 = pj == 1
        m2 = pj == 2
        hj = jnp.where(m0, h_ch[0],
             jnp.where(m1, h_ch[1],
             jnp.where(m2, h_ch[2], h_ch[3])))
        cj = jnp.where(m0, c_ch[0],
             jnp.where(m1, c_ch[1],
             jnp.where(m2, c_ch[2], c_ch[3])))
        nh_cols.append(hj)
        nc_cols.append(cj)
    nh_perm = jnp.concatenate(nh_cols, axis=1)

    fg = jnp.dot(nh_perm, wf_ref[...],
                 preferred_element_type=jnp.float32) + bf_ref[...]
    iou = jnp.dot(nh_perm, wa_ref[...],
                  preferred_element_type=jnp.float32) + ba_ref[...]

    two_f_in = 2.0 * f_in
    c = jnp.zeros_like(f_in)
    for j in range(_NCH):
        f_j = jax.nn.sigmoid(fg[:, j * _HS:(j + 1) * _HS] + two_f_in)
        c = c + f_j * nc_cols[j]

    h_ref[...] = iou * jnp.tanh(c)
    c_ref[...] = c


@functools.partial(jax.jit, static_argnames=())
def kernel(x, x_mask, neighbour_h, neighbour_c, pos,
           W_fin, b_fin, W_f, b_f, W_aggr, b_aggr):
    n, xs = x.shape
    _, nch, hs = neighbour_h.shape
    fw = nch * hs

    block = 2000
    grid = (pl.cdiv(n, block),)

    xm2 = x_mask.reshape(n, 1)

    row = lambda i: (i, 0)
    row3 = lambda i: (i, 0, 0)
    rep = lambda i: (0, 0)

    h, c = pl.pallas_call(
        _cell_kernel,
        grid=grid,
        in_specs=[
            pl.BlockSpec((block, xs), row),
            pl.BlockSpec((block, 1), row),
            pl.BlockSpec((block, nch, hs), row3),
            pl.BlockSpec((block, nch, hs), row3),
            pl.BlockSpec((block, nch), row),
            pl.BlockSpec((xs, hs), rep),
            pl.BlockSpec((1, hs), rep),
            pl.BlockSpec((fw, fw), rep),
            pl.BlockSpec((1, fw), rep),
            pl.BlockSpec((fw, hs), rep),
            pl.BlockSpec((1, hs), rep),
        ],
        out_specs=[
            pl.BlockSpec((block, hs), row),
            pl.BlockSpec((block, hs), row),
        ],
        out_shape=[
            jax.ShapeDtypeStruct((n, hs), jnp.float32),
            jax.ShapeDtypeStruct((n, hs), jnp.float32),
        ],
    )(x, xm2, neighbour_h, neighbour_c, pos,
      W_fin, b_fin.reshape(1, hs), W_f, b_f.reshape(1, fw),
      W_aggr, b_aggr.reshape(1, hs))
    return h, c


# speedup vs baseline: 1.6499x; 1.4239x over previous
import functools

import jax
import jax.numpy as jnp
from jax.experimental import pallas as pl

_NCH = 4
_HS = 128


def _cell_kernel(x_ref, xm_ref, nh_ref, nc_ref, pos_ref,
                 wfin_ref, bfin_ref, wf_ref, bf_ref, wa_ref, ba_ref,
                 h_ref, c_ref):
    x = x_ref[...]
    xm = xm_ref[...]
    pos = pos_ref[...]
    b = x.shape[0]

    f_in = (jnp.dot(x, wfin_ref[...], preferred_element_type=jnp.float32)
            + bfin_ref[...]) * xm

    nh = nh_ref[...].reshape(b, _NCH * _HS)
    nc = nc_ref[...].reshape(b, _NCH * _HS)
    h_ch = [nh[:, k * _HS:(k + 1) * _HS] for k in range(_NCH)]
    c_ch = [nc[:, k * _HS:(k + 1) * _HS] for k in range(_NCH)]

    nh_cols = []
    nc_cols = []
    for j in range(_NCH):
        pj = pos[:, j][:, None]
        m0 = pj == 0
        m1 = pj == 1
        m2 = pj == 2
        hj = jnp.where(m0, h_ch[0],
             jnp.where(m1, h_ch[1],
             jnp.where(m2, h_ch[2], h_ch[3])))
        cj = jnp.where(m0, c_ch[0],
             jnp.where(m1, c_ch[1],
             jnp.where(m2, c_ch[2], c_ch[3])))
        nh_cols.append(hj)
        nc_cols.append(cj)
    nh_perm = jnp.concatenate(nh_cols, axis=1)

    fg = jnp.dot(nh_perm, wf_ref[...],
                 preferred_element_type=jnp.float32) + bf_ref[...]
    iou = jnp.dot(nh_perm, wa_ref[...],
                  preferred_element_type=jnp.float32) + ba_ref[...]

    two_f_in = 2.0 * f_in
    c = jnp.zeros_like(f_in)
    for j in range(_NCH):
        f_j = jax.nn.sigmoid(fg[:, j * _HS:(j + 1) * _HS] + two_f_in)
        c = c + f_j * nc_cols[j]

    h_ref[...] = iou * jnp.tanh(c)
    c_ref[...] = c


@functools.partial(jax.jit, static_argnames=())
def kernel(x, x_mask, neighbour_h, neighbour_c, pos,
           W_fin, b_fin, W_f, b_f, W_aggr, b_aggr):
    n, xs = x.shape
    _, nch, hs = neighbour_h.shape
    fw = nch * hs

    block = 2000
    grid = (pl.cdiv(n, block),)

    xm2 = x_mask.reshape(n, 1)

    row = lambda i: (i, 0)
    row3 = lambda i: (i, 0, 0)
    rep = lambda i: (0, 0)

    h, c = pl.pallas_call(
        _cell_kernel,
        grid=grid,
        in_specs=[
            pl.BlockSpec((block, xs), row),
            pl.BlockSpec((block, 1), row),
            pl.BlockSpec((block, nch, hs), row3),
            pl.BlockSpec((block, nch, hs), row3),
            pl.BlockSpec((block, nch), row),
            pl.BlockSpec((xs, hs), rep),
            pl.BlockSpec((1, hs), rep),
            pl.BlockSpec((fw, fw), rep),
            pl.BlockSpec((1, fw), rep),
            pl.BlockSpec((fw, hs), rep),
            pl.BlockSpec((1, hs), rep),
        ],
        out_specs=[
            pl.BlockSpec((block, hs), row),
            pl.BlockSpec((block, hs), row),
        ],
        out_shape=[
            jax.ShapeDtypeStruct((n, hs), jnp.float32),
            jax.ShapeDtypeStruct((n, hs), jnp.float32),
        ],
    )(x, xm2, neighbour_h, neighbour_c, pos,
      W_fin, b_fin.reshape(1, hs), W_f, b_f.reshape(1, fw),
      W_aggr, b_aggr.reshape(1, hs))
    return h, c
